# baseline (speedup 1.0000x reference)
import functools

import jax
import jax.numpy as jnp
import numpy as np
from jax import lax
from jax.experimental import pallas as pl
from jax.experimental.pallas import tpu as pltpu
from jax.experimental.pallas import tpu_sc as plsc

T = 200
NUM_GRAPHS = 512
D_FEAT = 128
HIDDEN = 64

_INTERPRET = False

def _sched_table():
    tt = np.arange(T + 1, dtype=np.float64)
    alpha_bar = (1.0 - (tt / T) ** 2.0) ** 2
    alpha_bar = np.clip(alpha_bar, 1e-4, 1.0)
    alpha = np.clip(alpha_bar[1:] / alpha_bar[:-1], 1e-3, 1.0)
    alpha_bar = np.cumprod(alpha)
    sab = np.sqrt(alpha_bar)
    somab = np.sqrt(1.0 - alpha_bar)
    out = np.zeros((256, 2), np.float32)
    out[:T, 0] = sab
    out[:T, 1] = somab
    return out

_SCHED_NP = _sched_table()

_HI = jax.lax.Precision.DEFAULT


def _dot(a, b):
    return jnp.dot(a, b, precision=_HI, preferred_element_type=jnp.float32)


def _segsum_body(segs_ref, p8_ref, out_ref):
    i = pl.program_id(0)
    seg_row = segs_ref[...].reshape(1, -1)
    iota_g = jax.lax.broadcasted_iota(jnp.int32, (NUM_GRAPHS, 1), 0).astype(jnp.float32)
    onehot_t = (iota_g == seg_row).astype(jnp.float32)
    part = _dot(onehot_t, p8_ref[...])

    @pl.when(i == 0)
    def _init():
        out_ref[...] = part[None]

    @pl.when(i != 0)
    def _acc():
        out_ref[...] = out_ref[...] + part[None]


def _segment_sums_tc(p8, segs3, block):
    n = p8.shape[0]
    nb = n // block
    return pl.pallas_call(
        _segsum_body,
        grid=(nb,),
        in_specs=[
            pl.BlockSpec((1, 1, block), lambda i: (i, 0, 0)),
            pl.BlockSpec((block, 8), lambda i: (i, 0)),
        ],
        out_specs=pl.BlockSpec((1, NUM_GRAPHS, 8), lambda i: (0, 0, 0)),
        out_shape=jax.ShapeDtypeStruct((1, NUM_GRAPHS, 8), jnp.float32),
        compiler_params=pltpu.CompilerParams(
            dimension_semantics=("arbitrary",)),
        interpret=_INTERPRET,
    )(segs3, p8)


_NSC = 2
_NSS = 16
_NW = _NSC * _NSS
_CHUNK = 3200
_NPAD = _NW * _CHUNK
_SROWS = 520
_JB = 128
_NJ = _CHUNK // _JB

_SC_MESH = dict(core_axis_name="c", subcore_axis_name="s")


_NCOL = 7


def _sc_segsum(p7f, seg2):
    @functools.partial(
        pl.kernel,
        out_type=jax.ShapeDtypeStruct((_NW, _SROWS * 8), jnp.float32),
        mesh=plsc.VectorSubcoreMesh(**_SC_MESH),
        compiler_params=pltpu.CompilerParams(needs_layout_passes=False),
        scratch_types=[pltpu.VMEM((_CHUNK,), jnp.float32)] * _NCOL
        + [pltpu.VMEM((_CHUNK,), jnp.int32),
           pltpu.VMEM((_SROWS * 8,), jnp.float32)],
    )
    def k(p7f_hbm, seg_hbm, out_hbm, *refs):
        bufs = refs[:_NCOL]
        idxf = refs[_NCOL]
        tbl = refs[_NCOL + 1]
        cid = lax.axis_index("c")
        sid = lax.axis_index("s")
        wid = cid * _NSS + sid

        for c in range(_NCOL):
            pltpu.sync_copy(
                p7f_hbm.at[pl.ds(c * _NPAD + wid * _CHUNK, _CHUNK)],
                bufs[c])
        pltpu.sync_copy(seg_hbm.at[wid], idxf)

        z = jnp.zeros((16,), jnp.float32)

        def zero(k_, carry):
            tbl[pl.ds(k_ * 16, 16)] = z
            return carry

        lax.fori_loop(0, (_SROWS * 8) // 16, zero, 0)

        def body(g, carry):
            seg16 = idxf[pl.ds(g * 16, 16)]
            base = seg16 * 8
            for c in range(_NCOL):
                v = bufs[c][pl.ds(g * 16, 16)]
                plsc.addupdate_scatter(tbl, [base + c], v)
            return carry

        lax.fori_loop(0, _CHUNK // 16, body, 0)

        pltpu.sync_copy(tbl, out_hbm.at[wid])

    return k(p7f, seg2)


def _finalize_body(parts_ref, t_ref, sched_ref, tbl_ref):
    sums = jnp.sum(parts_ref[...], axis=0)
    cnt = jnp.maximum(sums[:, 6:7], 1.0)
    means = sums[:, 0:6] / cnt
    t_f = t_ref[...].astype(jnp.float32)
    iota_t = jax.lax.broadcasted_iota(jnp.int32, (1, 256), 1).astype(jnp.float32)
    onehot_t = (t_f == iota_t).astype(jnp.float32)
    sch = _dot(onehot_t, sched_ref[...])
    tfeat = t_f * (1.0 / T)
    pad = jnp.zeros((_SROWS, 7), jnp.float32)
    tbl_ref[...] = jnp.concatenate([means, sch, tfeat, pad], axis=1)


def _finalize_tc(parts, tpad2, sched):
    full = lambda *s: pl.BlockSpec(s, lambda: (0,) * len(s))
    return pl.pallas_call(
        _finalize_body,
        in_specs=[full(_NW, _SROWS, 8), full(_SROWS, 1), full(256, 2)],
        out_specs=full(_SROWS, 16),
        out_shape=jax.ShapeDtypeStruct((_SROWS, 16), jnp.float32),
        interpret=_INTERPRET,
    )(parts, tpad2, sched)


def _sc_expand(tbl16, seg3, n):
    @functools.partial(
        pl.kernel,
        out_type=jax.ShapeDtypeStruct((n, 16), jnp.float32),
        mesh=plsc.VectorSubcoreMesh(**_SC_MESH),
        scratch_types=[
            pltpu.VMEM((_NJ, _JB), jnp.int32),
            pltpu.VMEM((_JB, 16), jnp.float32),
            pltpu.SemaphoreType.DMA,
        ],
    )
    def k(tbl_hbm, seg_hbm, out_hbm, idxv, rows_v, sem):
        cid = lax.axis_index("c")
        sid = lax.axis_index("s")
        wid = cid * _NSS + sid

        pltpu.sync_copy(seg_hbm.at[wid], idxv)

        last_rows = n - (_NW - 1) * _CHUNK
        tail = last_rows % _JB
        my_full = jnp.where(wid == _NW - 1, last_rows // _JB, _NJ)

        def body(j, carry):
            pltpu.async_copy(tbl_hbm.at[idxv.at[j]], rows_v, sem).wait()
            pltpu.sync_copy(
                rows_v, out_hbm.at[pl.ds(wid * _CHUNK + j * _JB, _JB)])
            return carry

        lax.fori_loop(0, my_full, body, 0)

        if tail:
            @pl.when(wid == _NW - 1)
            def _tail():
                pltpu.async_copy(tbl_hbm.at[idxv.at[my_full]], rows_v,
                                 sem).wait()
                pltpu.sync_copy(
                    rows_v.at[pl.ds(0, tail)],
                    out_hbm.at[pl.ds(wid * _CHUNK + last_rows - tail, tail)])

    return k(tbl16, seg3)


def _main_body(tbl_ref, w1a_ref, w1b_ref, b1_ref,
               w2x_ref, b2x_ref, w2p_ref, b2p_ref,
               lx_ref, xe_ref, p8_ref,
               ox_ref, op_ref):
    i = pl.program_id(0)

    p8 = p8_ref[...]
    seg_col = p8[:, 7:8]
    iota_g = jax.lax.broadcasted_iota(
        jnp.int32, (1, _SROWS), 1).astype(jnp.float32)
    onehot = (seg_col == iota_g).astype(jnp.float32)
    vals = _dot(onehot, tbl_ref[...])

    mean_pos = vals[:, 0:3]
    mean_eps = vals[:, 3:6]
    sab = vals[:, 6:7]
    somab = vals[:, 7:8]
    tfeat = vals[:, 8:9]

    pos = p8[:, 0:3]
    eps = p8[:, 3:6]
    pos_eps = eps - mean_eps
    x_t_pos = sab * (pos - mean_pos) + somab * pos_eps
    xtp4 = jnp.concatenate([x_t_pos, tfeat], axis=1)

    xe = xe_ref[...]
    x_t_x = sab * lx_ref[...] + somab * xe

    pre = _dot(x_t_x, w1a_ref[...]) + _dot(xtp4, w1b_ref[...]) + b1_ref[...]
    h = jnp.maximum(pre, 0.0)

    xp = _dot(h, w2x_ref[...]) + b2x_ref[...]
    pp = _dot(h, w2p_ref[...]) + b2p_ref[...]

    ex = jnp.sum((xe - xp) ** 2)
    ep = jnp.sum((pos_eps - pp) ** 2)

    @pl.when(i == 0)
    def _init():
        ox_ref[...] = ex.reshape(1, 1)
        op_ref[...] = ep.reshape(1, 1)

    @pl.when(i != 0)
    def _acc():
        ox_ref[...] = ox_ref[...] + ex.reshape(1, 1)
        op_ref[...] = op_ref[...] + ep.reshape(1, 1)


def _main_pass(lx, xe, p8, tbl16, w1a, w1b4, b1, w2x, b2x, w2p, b2p,
               block):
    n = lx.shape[0]
    nb = n // block
    full = lambda *s: pl.BlockSpec(s, lambda i: (0,) * len(s))
    return pl.pallas_call(
        _main_body,
        grid=(nb,),
        in_specs=[
            full(_SROWS, 16),
            full(D_FEAT, HIDDEN),
            full(4, HIDDEN),
            full(1, HIDDEN),
            full(HIDDEN, D_FEAT),
            full(1, D_FEAT),
            full(HIDDEN, 3),
            full(1, 3),
            pl.BlockSpec((block, D_FEAT), lambda i: (i, 0)),
            pl.BlockSpec((block, D_FEAT), lambda i: (i, 0)),
            pl.BlockSpec((block, 8), lambda i: (i, 0)),
        ],
        out_specs=[
            pl.BlockSpec((1, 1), lambda i: (0, 0)),
            pl.BlockSpec((1, 1), lambda i: (0, 0)),
        ],
        out_shape=[
            jax.ShapeDtypeStruct((1, 1), jnp.float32),
            jax.ShapeDtypeStruct((1, 1), jnp.float32),
        ],
        compiler_params=pltpu.CompilerParams(
            dimension_semantics=("arbitrary",)),
        interpret=_INTERPRET,
    )(tbl16, w1a, w1b4, b1, w2x, b2x, w2p, b2p, lx, xe, p8)


BLOCK = 5000


def kernel(ligand_x, ligand_pos, protein_x, protein_pos, x_eps, pos_eps_raw,
           W1, b1, W2x, b2x, W2pos, b2pos, ligand_batch, protein_batch, t):
    n = ligand_x.shape[0]
    segf = ligand_batch.astype(jnp.float32)[:, None]
    ones = jnp.ones((n, 1), jnp.float32)
    p8 = jnp.concatenate([ligand_pos, pos_eps_raw, ones, segf], axis=1)

    npad = _NPAD - n
    p7t = jnp.concatenate(
        [p8[:, :_NCOL].T,
         jnp.zeros((_NCOL, npad), jnp.float32)], axis=1)
    seg_pad = jnp.concatenate(
        [ligand_batch.astype(jnp.int32),
         jnp.full((npad,), NUM_GRAPHS, jnp.int32)])
    seg2 = seg_pad.reshape(_NW, _CHUNK)
    tpad = jnp.concatenate(
        [t[:, 0].astype(jnp.int32),
         jnp.zeros((_SROWS - NUM_GRAPHS,), jnp.int32)])
    sched = jnp.asarray(_SCHED_NP)

    parts = _sc_segsum(p7t.reshape(-1), seg2)
    tbl16 = _finalize_tc(parts.reshape(_NW, _SROWS, 8),
                         tpad[:, None], sched)

    w1a = W1[0:D_FEAT]
    w1b4 = W1[D_FEAT:D_FEAT + 4]
    ox, op = _main_pass(ligand_x, x_eps, p8, tbl16,
                        w1a, w1b4, b1[None, :], W2x, b2x[None, :],
                        W2pos, b2pos[None, :], BLOCK)

    sum_x = ox[0, 0]
    sum_pos = op[0, 0]
    L_x = sum_x / (n * D_FEAT)
    L_pos = sum_pos / (n * 3)
    L_simple = 0.25 * (L_pos + L_x)
    L_unweighted = 0.5 * (sum_x + sum_pos) / (n * (D_FEAT + 3))
    return (L_simple, L_unweighted, L_pos, L_x)
